# Initial kernel scaffold; baseline (speedup 1.0000x reference)
import functools

import jax
import jax.numpy as jnp
from jax import lax
from jax.experimental import pallas as pl
from jax.experimental.pallas import tpu as pltpu
from jax.experimental.pallas import tpu_sc as plsc

N = 10000
E = 320000
DIN = 128
EDIM = 16
HID = 32
H1 = 4
OUT = 128

NC = 2
NS = 16
NW = NC * NS
B = 128
EP = ((E + NW * B - 1) // (NW * B)) * (NW * B)
NPAD = ((N + 1 + NS - 1) // NS) * NS
ACCW = 144
BM = 1000


def _blockdiag(a):
    h, c = a.shape
    out = jnp.zeros((h * c, h), jnp.float32)
    return out.at[jnp.arange(h * c), jnp.repeat(jnp.arange(h), c)].set(
        a.reshape(-1))


def _node1_body(x_ref, w1_ref, as_ref, ad_ref, h_ref, ts_ref, td_ref):
    h = jnp.dot(x_ref[...], w1_ref[...], preferred_element_type=jnp.float32)
    h_ref[...] = h
    s = jnp.dot(h, as_ref[...], preferred_element_type=jnp.float32)
    d = jnp.dot(h, ad_ref[...], preferred_element_type=jnp.float32)
    z = jnp.zeros((s.shape[0], 12), jnp.float32)
    ts_ref[...] = jnp.concatenate([s, z], axis=1)
    td_ref[...] = jnp.concatenate([d, z], axis=1)


def _node1_call(x, W1, As1, Ad1):
    nb = N // BM
    full = lambda shape: pl.BlockSpec(shape, lambda i: (0, 0))
    row = lambda w: pl.BlockSpec((BM, w), lambda i: (i, 0))
    return pl.pallas_call(
        _node1_body,
        grid=(nb,),
        in_specs=[row(DIN), full((DIN, H1 * HID)), full((H1 * HID, H1)),
                  full((H1 * HID, H1))],
        out_specs=[row(H1 * HID), row(16), row(16)],
        out_shape=[jax.ShapeDtypeStruct((N, H1 * HID), jnp.float32),
                   jax.ShapeDtypeStruct((N, 16), jnp.float32),
                   jax.ShapeDtypeStruct((N, 16), jnp.float32)],
    )(x, W1, As1, Ad1)


def _edge_body(ea_ref, we1_ref, ae1_ref, we2_ref, ae2_ref, et_ref):
    ve1 = jnp.dot(we1_ref[...], ae1_ref[...],
                  preferred_element_type=jnp.float32)
    ve2 = jnp.dot(we2_ref[...], ae2_ref[...],
                  preferred_element_type=jnp.float32)
    ea = ea_ref[...]
    e1 = jnp.dot(ea, ve1, preferred_element_type=jnp.float32)
    e2 = jnp.dot(ea, ve2, preferred_element_type=jnp.float32)
    bm = ea.shape[0]
    one = jnp.ones((bm, 1), jnp.float32)
    z6 = jnp.zeros((bm, 6), jnp.float32)
    et_ref[...] = jnp.concatenate([e1, e1, e2, one, z6], axis=1)


def _edge_call(eap, We1, Ae1, We2, Ae2):
    bm = NW * B
    nb = EP // bm
    full = lambda shape: pl.BlockSpec(shape, lambda i: (0, 0))
    return pl.pallas_call(
        _edge_body,
        grid=(nb,),
        in_specs=[pl.BlockSpec((bm, EDIM), lambda i: (i, 0)),
                  full((EDIM, H1 * HID)), full((H1 * HID, H1)),
                  full((EDIM, OUT)), full((OUT, 1))],
        out_specs=pl.BlockSpec((bm, 16), lambda i: (i, 0)),
        out_shape=jax.ShapeDtypeStruct((EP, 16), jnp.float32),
    )(eap, We1, Ae1, We2, Ae2)


def _lrelu(v):
    return jnp.where(v >= 0, v, 0.2 * v)


def _mid_body(a0_ref, a1_ref, h1_ref, ts_ref, td_ref, w2_ref, as2_ref,
              ad2_ref, b1_ref, h2_ref, ts2_ref, td2_ref, aux_ref):
    acc = a0_ref[...] + a1_ref[...]
    den1 = acc[:, 128:132]
    se1 = acc[:, 132:136]
    se2 = acc[:, 136:137]
    deg = acc[:, 137:138]
    mdeg = jnp.maximum(deg, 1.0)
    s1 = ts_ref[...][:, 0:4]
    d1 = td_ref[...][:, 0:4]
    exs1 = jnp.exp(_lrelu(s1 + d1 + se1 / mdeg))
    dent = den1 + exs1 + 1e-16
    h1 = h1_ref[...]
    cols = []
    for hh in range(H1):
        sl = slice(hh * HID, (hh + 1) * HID)
        cols.append((acc[:, sl] + exs1[:, hh:hh + 1] * h1[:, sl])
                    / dent[:, hh:hh + 1])
    out1 = jnp.concatenate(cols, axis=1) + b1_ref[...]
    h2in = jnp.maximum(out1, 0.0)
    h2 = jnp.dot(h2in, w2_ref[...], preferred_element_type=jnp.float32)
    h2_ref[...] = h2
    s2 = jnp.dot(h2, as2_ref[...], preferred_element_type=jnp.float32)
    d2 = jnp.dot(h2, ad2_ref[...], preferred_element_type=jnp.float32)
    exs2 = jnp.exp(_lrelu(s2 + d2 + se2 / mdeg))
    bm = acc.shape[0]
    z8 = jnp.zeros((bm, 8), jnp.float32)
    z7 = jnp.zeros((bm, 7), jnp.float32)
    ts2_ref[...] = jnp.concatenate([z8, s2, z7], axis=1)
    td2_ref[...] = jnp.concatenate([z8, d2, z7], axis=1)
    aux_ref[...] = jnp.concatenate([exs2, z8, z7], axis=1)


def _mid_call(acc0, acc1, h1, ts, td, W2, As2, Ad2, b1row):
    nb = N // BM
    full = lambda shape: pl.BlockSpec(shape, lambda i: (0, 0))
    row = lambda w: pl.BlockSpec((BM, w), lambda i: (i, 0))
    return pl.pallas_call(
        _mid_body,
        grid=(nb,),
        in_specs=[row(ACCW), row(ACCW), row(H1 * HID), row(16), row(16),
                  full((H1 * HID, OUT)), full((OUT, 1)), full((OUT, 1)),
                  full((1, H1 * HID))],
        out_specs=[row(OUT), row(16), row(16), row(16)],
        out_shape=[jax.ShapeDtypeStruct((N, OUT), jnp.float32),
                   jax.ShapeDtypeStruct((N, 16), jnp.float32),
                   jax.ShapeDtypeStruct((N, 16), jnp.float32),
                   jax.ShapeDtypeStruct((N, 16), jnp.float32)],
    )(acc0, acc1, h1, ts, td, W2, As2, Ad2, b1row)


def _final_body(a0_ref, a1_ref, h2_ref, aux_ref, b2_ref, out_ref):
    acc = a0_ref[...] + a1_ref[...]
    den2 = acc[:, 136:137]
    exs2 = aux_ref[...][:, 0:1]
    h2 = h2_ref[...]
    out_ref[...] = ((acc[:, 0:128] + exs2 * h2)
                    / (den2 + exs2 + 1e-16)) + b2_ref[...]


def _final_call(acc0, acc1, h2, aux, b2row):
    nb = N // BM
    full = lambda shape: pl.BlockSpec(shape, lambda i: (0, 0))
    row = lambda w: pl.BlockSpec((BM, w), lambda i: (i, 0))
    return pl.pallas_call(
        _final_body,
        grid=(nb,),
        in_specs=[row(ACCW), row(ACCW), row(OUT), row(16), full((1, OUT))],
        out_specs=row(OUT),
        out_shape=jax.ShapeDtypeStruct((N, OUT), jnp.float32),
    )(acc0, acc1, h2, aux, b2row)


def _make_sc_pass(layer):
    mesh = plsc.VectorSubcoreMesh(core_axis_name="c", subcore_axis_name="s")
    nblocks = EP // B
    bpt = nblocks // NW
    rps = NPAD // NS

    @functools.partial(
        pl.kernel,
        out_type=jax.ShapeDtypeStruct((NC, NPAD, ACCW), jnp.float32),
        mesh=mesh,
        scratch_types=[
            pltpu.VMEM((B,), jnp.int32),
            pltpu.VMEM((B,), jnp.int32),
            pltpu.VMEM((B, 16), jnp.float32),
            pltpu.VMEM((B, 16), jnp.float32),
            pltpu.VMEM((B, 16), jnp.float32),
            pltpu.VMEM((B, 128), jnp.float32),
            pltpu.VMEM((B, ACCW), jnp.float32),
            pltpu.VMEM_SHARED((NPAD, ACCW), jnp.float32),
            pltpu.SemaphoreType.DMA,
            pltpu.SemaphoreType.DMA,
            pltpu.SemaphoreType.DMA,
        ],
    )
    def sc_pass(src_hbm, dst_hbm, et_hbm, tabs_hbm, tabd_hbm, h_hbm, zero_hbm,
                out_hbm, src_v, dst_v, s_v, d_v, et_v, h_v, msg_v, acc_sh,
                sem0, sem1, sem2):
        c = lax.axis_index("c")
        s = lax.axis_index("s")
        wid = s * NC + c

        pltpu.sync_copy(zero_hbm.at[pl.ds(s * rps, rps)],
                        acc_sh.at[pl.ds(s * rps, rps)])
        plsc.subcore_barrier()

        lanes = lax.iota(jnp.int32, 16)

        @pl.loop(0, bpt)
        def _block(bi):
            base = (bi * NW + wid) * B
            pltpu.sync_copy(src_hbm.at[pl.ds(base, B)], src_v)
            pltpu.sync_copy(dst_hbm.at[pl.ds(base, B)], dst_v)
            pltpu.sync_copy(et_hbm.at[pl.ds(base, B)], et_v)
            cp0 = pltpu.async_copy(tabs_hbm.at[src_v], s_v, sem0)
            cp1 = pltpu.async_copy(tabd_hbm.at[dst_v], d_v, sem1)
            cp2 = pltpu.async_copy(h_hbm.at[src_v], h_v, sem2)
            cp0.wait()
            cp1.wait()
            cp2.wait()

            @pl.loop(0, B)
            def _edge(i):
                arow = s_v[i] + d_v[i] + et_v[i]
                exrow = jnp.exp(_lrelu(arow))
                if layer == 1:
                    tail = jnp.where(lanes < 4, exrow, et_v[i])
                else:
                    tail = jnp.where(lanes == 8, exrow,
                                     jnp.zeros((16,), jnp.float32))
                msg_v[i, pl.ds(128, 16)] = tail
                for j in range(8):
                    lane = 128 + (j // 2 if layer == 1 else 8)
                    bc = jnp.full((16,), msg_v[i, lane], jnp.float32)
                    msg_v[i, pl.ds(j * 16, 16)] = (
                        h_v[i, pl.ds(j * 16, 16)] * bc)

            pltpu.sync_copy(msg_v, acc_sh.at[dst_v], add=True)

        plsc.subcore_barrier()
        pltpu.sync_copy(acc_sh.at[pl.ds(s * rps, rps)],
                        out_hbm.at[c].at[pl.ds(s * rps, rps)])

    return sc_pass


_sc_pass1 = _make_sc_pass(1)
_sc_pass2 = _make_sc_pass(2)


def kernel(x, edge_index, edge_attr, W1, a_src1, a_dst1, We1, ae1, b1,
           W2, a_src2, a_dst2, We2, ae2, b2):
    src, dst = edge_index[0], edge_index[1]

    As1 = _blockdiag(a_src1)
    Ad1 = _blockdiag(a_dst1)
    Ae1 = _blockdiag(ae1)
    As2 = a_src2.reshape(OUT, 1)
    Ad2 = a_dst2.reshape(OUT, 1)
    Ae2 = ae2.reshape(OUT, 1)

    srcp = jnp.concatenate([src, jnp.zeros((EP - E,), jnp.int32)])
    dstp = jnp.concatenate([dst, jnp.full((EP - E,), N, jnp.int32)])
    eap = jnp.concatenate(
        [edge_attr, jnp.zeros((EP - E, EDIM), jnp.float32)], axis=0)

    et = _edge_call(eap, We1, Ae1, We2, Ae2)
    h1, ts1, td1 = _node1_call(x, W1, As1, Ad1)

    zeros_acc = jnp.zeros((NPAD, ACCW), jnp.float32)
    padn = lambda t: jnp.concatenate(
        [t, jnp.zeros((NPAD - N, t.shape[1]), t.dtype)], axis=0)

    acc1 = _sc_pass1(srcp, dstp, et, padn(ts1), padn(td1), padn(h1),
                     zeros_acc)
    h2, ts2, td2, aux = _mid_call(acc1[0, :N], acc1[1, :N], h1, ts1, td1,
                                  W2, As2, Ad2, b1.reshape(1, -1))

    acc2 = _sc_pass2(srcp, dstp, et, padn(ts2), padn(td2), padn(h2),
                     zeros_acc)
    out = _final_call(acc2[0, :N], acc2[1, :N], h2, aux, b2.reshape(1, -1))
    return out


# speedup vs baseline: 14.7947x; 14.7947x over previous
import functools

import jax
import jax.numpy as jnp
from jax import lax
from jax.experimental import pallas as pl
from jax.experimental.pallas import tpu as pltpu
from jax.experimental.pallas import tpu_sc as plsc

N = 10000
E = 320000
DIN = 128
EDIM = 16
HID = 32
H1 = 4
OUT = 128

NC = 2
NS = 16
NW = NC * NS
B = 64
EP = ((E + NW * B - 1) // (NW * B)) * (NW * B)
NPAD = ((N + 1 + NS * 8 - 1) // (NS * 8)) * (NS * 8)
ACCW = 144
BM = 1000


def _blockdiag(a):
    h, c = a.shape
    out = jnp.zeros((h * c, h), jnp.float32)
    return out.at[jnp.arange(h * c), jnp.repeat(jnp.arange(h), c)].set(
        a.reshape(-1))


def _node1_body(x_ref, w1_ref, as_ref, ad_ref, h_ref, ts_ref, td_ref):
    h = jnp.dot(x_ref[...], w1_ref[...], preferred_element_type=jnp.float32)
    h_ref[...] = h
    s = jnp.dot(h, as_ref[...], preferred_element_type=jnp.float32)
    d = jnp.dot(h, ad_ref[...], preferred_element_type=jnp.float32)
    z = jnp.zeros((s.shape[0], 12), jnp.float32)
    ts_ref[...] = jnp.concatenate([s, z], axis=1)
    td_ref[...] = jnp.concatenate([d, z], axis=1)


def _node1_call(x, W1, As1, Ad1):
    nb = N // BM
    full = lambda shape: pl.BlockSpec(shape, lambda i: (0, 0))
    row = lambda w: pl.BlockSpec((BM, w), lambda i: (i, 0))
    return pl.pallas_call(
        _node1_body,
        grid=(nb,),
        in_specs=[row(DIN), full((DIN, H1 * HID)), full((H1 * HID, H1)),
                  full((H1 * HID, H1))],
        out_specs=[row(H1 * HID), row(16), row(16)],
        out_shape=[jax.ShapeDtypeStruct((N, H1 * HID), jnp.float32),
                   jax.ShapeDtypeStruct((N, 16), jnp.float32),
                   jax.ShapeDtypeStruct((N, 16), jnp.float32)],
    )(x, W1, As1, Ad1)


def _edge_body(ea_ref, we1_ref, ae1_ref, we2_ref, ae2_ref, et_ref):
    ve1 = jnp.dot(we1_ref[...], ae1_ref[...],
                  preferred_element_type=jnp.float32)
    ve2 = jnp.dot(we2_ref[...], ae2_ref[...],
                  preferred_element_type=jnp.float32)
    ea = ea_ref[...]
    e1 = jnp.dot(ea, ve1, preferred_element_type=jnp.float32)
    e2 = jnp.dot(ea, ve2, preferred_element_type=jnp.float32)
    bm = ea.shape[0]
    one = jnp.ones((bm, 1), jnp.float32)
    z6 = jnp.zeros((bm, 6), jnp.float32)
    et_ref[...] = jnp.concatenate([e1, e1, e2, one, z6], axis=1)


def _edge_call(eap, We1, Ae1, We2, Ae2):
    bm = NW * B
    nb = EP // bm
    full = lambda shape: pl.BlockSpec(shape, lambda i: (0, 0))
    return pl.pallas_call(
        _edge_body,
        grid=(nb,),
        in_specs=[pl.BlockSpec((bm, EDIM), lambda i: (i, 0)),
                  full((EDIM, H1 * HID)), full((H1 * HID, H1)),
                  full((EDIM, OUT)), full((OUT, 1))],
        out_specs=pl.BlockSpec((bm, 16), lambda i: (i, 0)),
        out_shape=jax.ShapeDtypeStruct((EP, 16), jnp.float32),
    )(eap, We1, Ae1, We2, Ae2)


def _lrelu(v):
    return jnp.where(v >= 0, v, 0.2 * v)


def _mid_body(a0_ref, a1_ref, h1_ref, ts_ref, td_ref, w2_ref, as2_ref,
              ad2_ref, b1_ref, h2_ref, ts2_ref, td2_ref, aux_ref):
    acc = a0_ref[...] + a1_ref[...]
    den1 = acc[:, 128:132]
    se1 = acc[:, 132:136]
    se2 = acc[:, 136:137]
    deg = acc[:, 137:138]
    mdeg = jnp.maximum(deg, 1.0)
    s1 = ts_ref[...][:, 0:4]
    d1 = td_ref[...][:, 0:4]
    exs1 = jnp.exp(_lrelu(s1 + d1 + se1 / mdeg))
    dent = den1 + exs1 + 1e-16
    h1 = h1_ref[...]
    cols = []
    for hh in range(H1):
        sl = slice(hh * HID, (hh + 1) * HID)
        cols.append((acc[:, sl] + exs1[:, hh:hh + 1] * h1[:, sl])
                    / dent[:, hh:hh + 1])
    out1 = jnp.concatenate(cols, axis=1) + b1_ref[...]
    h2in = jnp.maximum(out1, 0.0)
    h2 = jnp.dot(h2in, w2_ref[...], preferred_element_type=jnp.float32)
    h2_ref[...] = h2
    s2 = jnp.dot(h2, as2_ref[...], preferred_element_type=jnp.float32)
    d2 = jnp.dot(h2, ad2_ref[...], preferred_element_type=jnp.float32)
    exs2 = jnp.exp(_lrelu(s2 + d2 + se2 / mdeg))
    bm = acc.shape[0]
    z8 = jnp.zeros((bm, 8), jnp.float32)
    z7 = jnp.zeros((bm, 7), jnp.float32)
    ts2_ref[...] = jnp.concatenate([z8, s2, z7], axis=1)
    td2_ref[...] = jnp.concatenate([z8, d2, z7], axis=1)
    aux_ref[...] = jnp.concatenate([exs2, z8, z7], axis=1)


def _mid_call(acc0, acc1, h1, ts, td, W2, As2, Ad2, b1row):
    nb = N // BM
    full = lambda shape: pl.BlockSpec(shape, lambda i: (0, 0))
    row = lambda w: pl.BlockSpec((BM, w), lambda i: (i, 0))
    return pl.pallas_call(
        _mid_body,
        grid=(nb,),
        in_specs=[row(ACCW), row(ACCW), row(H1 * HID), row(16), row(16),
                  full((H1 * HID, OUT)), full((OUT, 1)), full((OUT, 1)),
                  full((1, H1 * HID))],
        out_specs=[row(OUT), row(16), row(16), row(16)],
        out_shape=[jax.ShapeDtypeStruct((N, OUT), jnp.float32),
                   jax.ShapeDtypeStruct((N, 16), jnp.float32),
                   jax.ShapeDtypeStruct((N, 16), jnp.float32),
                   jax.ShapeDtypeStruct((N, 16), jnp.float32)],
    )(acc0, acc1, h1, ts, td, W2, As2, Ad2, b1row)


def _final_body(a0_ref, a1_ref, h2_ref, aux_ref, b2_ref, out_ref):
    acc = a0_ref[...] + a1_ref[...]
    den2 = acc[:, 136:137]
    exs2 = aux_ref[...][:, 0:1]
    h2 = h2_ref[...]
    out_ref[...] = ((acc[:, 0:128] + exs2 * h2)
                    / (den2 + exs2 + 1e-16)) + b2_ref[...]


def _final_call(acc0, acc1, h2, aux, b2row):
    nb = N // BM
    full = lambda shape: pl.BlockSpec(shape, lambda i: (0, 0))
    row = lambda w: pl.BlockSpec((BM, w), lambda i: (i, 0))
    return pl.pallas_call(
        _final_body,
        grid=(nb,),
        in_specs=[row(ACCW), row(ACCW), row(OUT), row(16), full((1, OUT))],
        out_specs=row(OUT),
        out_shape=jax.ShapeDtypeStruct((N, OUT), jnp.float32),
    )(acc0, acc1, h2, aux, b2row)


def _make_sc_pass(layer):
    mesh = plsc.VectorSubcoreMesh(core_axis_name="c", subcore_axis_name="s")
    nblocks = EP // B
    bpt = nblocks // NW
    rps = NPAD // NS

    @functools.partial(
        pl.kernel,
        out_type=jax.ShapeDtypeStruct((NC, NPAD, ACCW), jnp.float32),
        mesh=mesh,
        compiler_params=pltpu.CompilerParams(use_tc_tiling_on_sc=False),
        scratch_types=[
            pltpu.VMEM((B,), jnp.int32),
            pltpu.VMEM((B,), jnp.int32),
            pltpu.VMEM((B, 16), jnp.float32),
            pltpu.VMEM((B, 16), jnp.float32),
            pltpu.VMEM((B, 16), jnp.float32),
            pltpu.VMEM((B, 128), jnp.float32),
            pltpu.VMEM((B, ACCW), jnp.float32),
            pltpu.VMEM_SHARED((NPAD, ACCW), jnp.float32),
            pltpu.SemaphoreType.DMA,
            pltpu.SemaphoreType.DMA,
            pltpu.SemaphoreType.DMA,
        ],
    )
    def sc_pass(src_hbm, dst_hbm, et_hbm, tabs_hbm, tabd_hbm, h_hbm, zero_hbm,
                out_hbm, src_v, dst_v, s_v, d_v, et_v, h_v, msg_v, acc_sh,
                sem0, sem1, sem2):
        c = lax.axis_index("c")
        s = lax.axis_index("s")
        wid = s * NC + c

        pltpu.sync_copy(zero_hbm.at[pl.ds(s * rps, rps)],
                        acc_sh.at[pl.ds(s * rps, rps)])
        plsc.subcore_barrier()

        lanes = lax.iota(jnp.int32, 16)

        @pl.loop(0, bpt)
        def _block(bi):
            base = (bi * NW + wid) * B
            pltpu.sync_copy(src_hbm.at[pl.ds(base, B)], src_v)
            pltpu.sync_copy(dst_hbm.at[pl.ds(base, B)], dst_v)
            pltpu.sync_copy(et_hbm.at[pl.ds(base, B)], et_v)
            cp0 = pltpu.async_copy(tabs_hbm.at[src_v], s_v, sem0)
            cp1 = pltpu.async_copy(tabd_hbm.at[dst_v], d_v, sem1)
            cp2 = pltpu.async_copy(h_hbm.at[src_v], h_v, sem2)
            cp0.wait()
            cp1.wait()
            cp2.wait()

            @pl.loop(0, B)
            def _edge(i):
                arow = s_v[i] + d_v[i] + et_v[i]
                exrow = jnp.exp(_lrelu(arow))
                if layer == 1:
                    tail = jnp.where(lanes < 4, exrow, et_v[i])
                else:
                    tail = jnp.where(lanes == 8, exrow,
                                     jnp.zeros((16,), jnp.float32))
                msg_v[i, pl.ds(128, 16)] = tail
                for j in range(8):
                    lane = j // 2 if layer == 1 else 8
                    bc = jnp.full((16,), exrow[lane], jnp.float32)
                    msg_v[i, pl.ds(j * 16, 16)] = (
                        h_v[i, pl.ds(j * 16, 16)] * bc)

            pltpu.sync_copy(msg_v, acc_sh.at[dst_v], add=True)

        plsc.subcore_barrier()
        pltpu.sync_copy(acc_sh.at[pl.ds(s * rps, rps)],
                        out_hbm.at[c].at[pl.ds(s * rps, rps)])

    return sc_pass


_sc_pass1 = _make_sc_pass(1)
_sc_pass2 = _make_sc_pass(2)


def kernel(x, edge_index, edge_attr, W1, a_src1, a_dst1, We1, ae1, b1,
           W2, a_src2, a_dst2, We2, ae2, b2):
    src, dst = edge_index[0], edge_index[1]

    As1 = _blockdiag(a_src1)
    Ad1 = _blockdiag(a_dst1)
    Ae1 = _blockdiag(ae1)
    As2 = a_src2.reshape(OUT, 1)
    Ad2 = a_dst2.reshape(OUT, 1)
    Ae2 = ae2.reshape(OUT, 1)

    srcp = jnp.concatenate([src, jnp.zeros((EP - E,), jnp.int32)])
    dstp = jnp.concatenate([dst, jnp.full((EP - E,), N, jnp.int32)])
    eap = jnp.concatenate(
        [edge_attr, jnp.zeros((EP - E, EDIM), jnp.float32)], axis=0)

    et = _edge_call(eap, We1, Ae1, We2, Ae2)
    h1, ts1, td1 = _node1_call(x, W1, As1, Ad1)

    zeros_acc = jnp.zeros((NPAD, ACCW), jnp.float32)
    padn = lambda t: jnp.concatenate(
        [t, jnp.zeros((NPAD - N, t.shape[1]), t.dtype)], axis=0)

    acc1 = _sc_pass1(srcp, dstp, et, padn(ts1), padn(td1), padn(h1),
                     zeros_acc)
    h2, ts2, td2, aux = _mid_call(acc1[0, :N], acc1[1, :N], h1, ts1, td1,
                                  W2, As2, Ad2, b1.reshape(1, -1))

    acc2 = _sc_pass2(srcp, dstp, et, padn(ts2), padn(td2), padn(h2),
                     zeros_acc)
    out = _final_call(acc2[0, :N], acc2[1, :N], h2, aux, b2.reshape(1, -1))
    return out


# speedup vs baseline: 15.5947x; 1.0541x over previous
import functools

import jax
import jax.numpy as jnp
from jax import lax
from jax.experimental import pallas as pl
from jax.experimental.pallas import tpu as pltpu
from jax.experimental.pallas import tpu_sc as plsc

N = 10000
E = 320000
DIN = 128
EDIM = 16
HID = 32
H1 = 4
OUT = 128

NC = 2
NS = 16
NW = NC * NS
B = 64
EP = ((E + NW * B * 2 - 1) // (NW * B * 2)) * (NW * B * 2)
NPAD = ((N + 1 + NS * 8 - 1) // (NS * 8)) * (NS * 8)
ACCW = 144
BM = 1000


def _blockdiag(a):
    h, c = a.shape
    out = jnp.zeros((h * c, h), jnp.float32)
    return out.at[jnp.arange(h * c), jnp.repeat(jnp.arange(h), c)].set(
        a.reshape(-1))


def _node1_body(x_ref, w1_ref, as_ref, ad_ref, h_ref, ts_ref, td_ref):
    h = jnp.dot(x_ref[...], w1_ref[...], preferred_element_type=jnp.float32)
    h_ref[...] = h
    s = jnp.dot(h, as_ref[...], preferred_element_type=jnp.float32)
    d = jnp.dot(h, ad_ref[...], preferred_element_type=jnp.float32)
    z = jnp.zeros((s.shape[0], 12), jnp.float32)
    ts_ref[...] = jnp.concatenate([s, z], axis=1)
    td_ref[...] = jnp.concatenate([d, z], axis=1)


def _node1_call(x, W1, As1, Ad1):
    nb = N // BM
    full = lambda shape: pl.BlockSpec(shape, lambda i: (0, 0))
    row = lambda w: pl.BlockSpec((BM, w), lambda i: (i, 0))
    return pl.pallas_call(
        _node1_body,
        grid=(nb,),
        in_specs=[row(DIN), full((DIN, H1 * HID)), full((H1 * HID, H1)),
                  full((H1 * HID, H1))],
        out_specs=[row(H1 * HID), row(16), row(16)],
        out_shape=[jax.ShapeDtypeStruct((N, H1 * HID), jnp.float32),
                   jax.ShapeDtypeStruct((N, 16), jnp.float32),
                   jax.ShapeDtypeStruct((N, 16), jnp.float32)],
    )(x, W1, As1, Ad1)


def _edge_body(ea_ref, we1_ref, ae1_ref, we2_ref, ae2_ref, et_ref):
    ve1 = jnp.dot(we1_ref[...], ae1_ref[...],
                  preferred_element_type=jnp.float32)
    ve2 = jnp.dot(we2_ref[...], ae2_ref[...],
                  preferred_element_type=jnp.float32)
    ea = ea_ref[...]
    e1 = jnp.dot(ea, ve1, preferred_element_type=jnp.float32)
    e2 = jnp.dot(ea, ve2, preferred_element_type=jnp.float32)
    bm = ea.shape[0]
    one = jnp.ones((bm, 1), jnp.float32)
    z6 = jnp.zeros((bm, 6), jnp.float32)
    et_ref[...] = jnp.concatenate([e1, e1, e2, one, z6], axis=1)


def _edge_call(eap, We1, Ae1, We2, Ae2):
    bm = NW * B
    nb = EP // bm
    full = lambda shape: pl.BlockSpec(shape, lambda i: (0, 0))
    return pl.pallas_call(
        _edge_body,
        grid=(nb,),
        in_specs=[pl.BlockSpec((bm, EDIM), lambda i: (i, 0)),
                  full((EDIM, H1 * HID)), full((H1 * HID, H1)),
                  full((EDIM, OUT)), full((OUT, 1))],
        out_specs=pl.BlockSpec((bm, 16), lambda i: (i, 0)),
        out_shape=jax.ShapeDtypeStruct((EP, 16), jnp.float32),
    )(eap, We1, Ae1, We2, Ae2)


def _lrelu(v):
    return jnp.where(v >= 0, v, 0.2 * v)


def _mid_body(a0_ref, a1_ref, h1_ref, ts_ref, td_ref, w2_ref, as2_ref,
              ad2_ref, b1_ref, h2_ref, ts2_ref, td2_ref, aux_ref):
    acc = a0_ref[...] + a1_ref[...]
    den1 = acc[:, 128:132]
    se1 = acc[:, 132:136]
    se2 = acc[:, 136:137]
    deg = acc[:, 137:138]
    mdeg = jnp.maximum(deg, 1.0)
    s1 = ts_ref[...][:, 0:4]
    d1 = td_ref[...][:, 0:4]
    exs1 = jnp.exp(_lrelu(s1 + d1 + se1 / mdeg))
    dent = den1 + exs1 + 1e-16
    h1 = h1_ref[...]
    cols = []
    for hh in range(H1):
        sl = slice(hh * HID, (hh + 1) * HID)
        cols.append((acc[:, sl] + exs1[:, hh:hh + 1] * h1[:, sl])
                    / dent[:, hh:hh + 1])
    out1 = jnp.concatenate(cols, axis=1) + b1_ref[...]
    h2in = jnp.maximum(out1, 0.0)
    h2 = jnp.dot(h2in, w2_ref[...], preferred_element_type=jnp.float32)
    h2_ref[...] = h2
    s2 = jnp.dot(h2, as2_ref[...], preferred_element_type=jnp.float32)
    d2 = jnp.dot(h2, ad2_ref[...], preferred_element_type=jnp.float32)
    exs2 = jnp.exp(_lrelu(s2 + d2 + se2 / mdeg))
    bm = acc.shape[0]
    z8 = jnp.zeros((bm, 8), jnp.float32)
    z7 = jnp.zeros((bm, 7), jnp.float32)
    ts2_ref[...] = jnp.concatenate([z8, s2, z7], axis=1)
    td2_ref[...] = jnp.concatenate([z8, d2, z7], axis=1)
    aux_ref[...] = jnp.concatenate([exs2, z8, z7], axis=1)


def _mid_call(acc0, acc1, h1, ts, td, W2, As2, Ad2, b1row):
    nb = N // BM
    full = lambda shape: pl.BlockSpec(shape, lambda i: (0, 0))
    row = lambda w: pl.BlockSpec((BM, w), lambda i: (i, 0))
    return pl.pallas_call(
        _mid_body,
        grid=(nb,),
        in_specs=[row(ACCW), row(ACCW), row(H1 * HID), row(16), row(16),
                  full((H1 * HID, OUT)), full((OUT, 1)), full((OUT, 1)),
                  full((1, H1 * HID))],
        out_specs=[row(OUT), row(16), row(16), row(16)],
        out_shape=[jax.ShapeDtypeStruct((N, OUT), jnp.float32),
                   jax.ShapeDtypeStruct((N, 16), jnp.float32),
                   jax.ShapeDtypeStruct((N, 16), jnp.float32),
                   jax.ShapeDtypeStruct((N, 16), jnp.float32)],
    )(acc0, acc1, h1, ts, td, W2, As2, Ad2, b1row)


def _final_body(a0_ref, a1_ref, h2_ref, aux_ref, b2_ref, out_ref):
    acc = a0_ref[...] + a1_ref[...]
    den2 = acc[:, 136:137]
    exs2 = aux_ref[...][:, 0:1]
    h2 = h2_ref[...]
    out_ref[...] = ((acc[:, 0:128] + exs2 * h2)
                    / (den2 + exs2 + 1e-16)) + b2_ref[...]


def _final_call(acc0, acc1, h2, aux, b2row):
    nb = N // BM
    full = lambda shape: pl.BlockSpec(shape, lambda i: (0, 0))
    row = lambda w: pl.BlockSpec((BM, w), lambda i: (i, 0))
    return pl.pallas_call(
        _final_body,
        grid=(nb,),
        in_specs=[row(ACCW), row(ACCW), row(OUT), row(16), full((1, OUT))],
        out_specs=row(OUT),
        out_shape=jax.ShapeDtypeStruct((N, OUT), jnp.float32),
    )(acc0, acc1, h2, aux, b2row)


def _make_sc_pass(layer):
    mesh = plsc.VectorSubcoreMesh(core_axis_name="c", subcore_axis_name="s")
    nblocks = EP // B
    bpt = nblocks // NW
    rps = NPAD // NS

    @functools.partial(
        pl.kernel,
        out_type=jax.ShapeDtypeStruct((NC, NPAD, ACCW), jnp.float32),
        mesh=mesh,
        compiler_params=pltpu.CompilerParams(use_tc_tiling_on_sc=False),
        scratch_types=[
            pltpu.VMEM((B,), jnp.int32), pltpu.VMEM((B,), jnp.int32),
            pltpu.VMEM((B,), jnp.int32), pltpu.VMEM((B,), jnp.int32),
            pltpu.VMEM((B, 16), jnp.float32), pltpu.VMEM((B, 16), jnp.float32),
            pltpu.VMEM((B, 16), jnp.float32), pltpu.VMEM((B, 16), jnp.float32),
            pltpu.VMEM((B, 16), jnp.float32), pltpu.VMEM((B, 16), jnp.float32),
            pltpu.VMEM((B, 128), jnp.float32),
            pltpu.VMEM((B, 128), jnp.float32),
            pltpu.VMEM((B, ACCW), jnp.float32),
            pltpu.VMEM_SHARED((NPAD, ACCW), jnp.float32),
            pltpu.SemaphoreType.DMA, pltpu.SemaphoreType.DMA,
            pltpu.SemaphoreType.DMA, pltpu.SemaphoreType.DMA,
            pltpu.SemaphoreType.DMA, pltpu.SemaphoreType.DMA,
        ],
    )
    def sc_pass(src_hbm, dst_hbm, et_hbm, tabs_hbm, tabd_hbm, h_hbm, zero_hbm,
                out_hbm, src0, src1, dst0, dst1, s0, s1, d0, d1, et0, et1,
                h0, h1b, msg_v, acc_sh, ss0, ss1, sd0, sd1, sh0, sh1):
        c = lax.axis_index("c")
        s = lax.axis_index("s")
        wid = s * NC + c
        src_v = [src0, src1]
        dst_v = [dst0, dst1]
        s_v = [s0, s1]
        d_v = [d0, d1]
        et_v = [et0, et1]
        h_v = [h0, h1b]
        sem_s = [ss0, ss1]
        sem_d = [sd0, sd1]
        sem_h = [sh0, sh1]

        pltpu.sync_copy(zero_hbm.at[pl.ds(s * rps, rps)],
                        acc_sh.at[pl.ds(s * rps, rps)])
        plsc.subcore_barrier()

        lanes = lax.iota(jnp.int32, 16)

        def fetch(b, bi):
            base = (bi * NW + wid) * B
            pltpu.sync_copy(src_hbm.at[pl.ds(base, B)], src_v[b])
            pltpu.sync_copy(dst_hbm.at[pl.ds(base, B)], dst_v[b])
            pltpu.sync_copy(et_hbm.at[pl.ds(base, B)], et_v[b])
            pltpu.make_async_copy(tabs_hbm.at[src_v[b]], s_v[b],
                                  sem_s[b]).start()
            pltpu.make_async_copy(tabd_hbm.at[dst_v[b]], d_v[b],
                                  sem_d[b]).start()
            pltpu.make_async_copy(h_hbm.at[src_v[b]], h_v[b],
                                  sem_h[b]).start()

        def finish(b):
            pltpu.make_async_copy(tabs_hbm.at[src_v[b]], s_v[b],
                                  sem_s[b]).wait()
            pltpu.make_async_copy(tabd_hbm.at[dst_v[b]], d_v[b],
                                  sem_d[b]).wait()
            pltpu.make_async_copy(h_hbm.at[src_v[b]], h_v[b],
                                  sem_h[b]).wait()

            @pl.loop(0, B)
            def _edge(i):
                arow = s_v[b][i] + d_v[b][i] + et_v[b][i]
                exrow = jnp.exp(_lrelu(arow))
                if layer == 1:
                    tail = jnp.where(lanes < 4, exrow, et_v[b][i])
                else:
                    tail = jnp.where(lanes == 8, exrow,
                                     jnp.zeros((16,), jnp.float32))
                msg_v[i, pl.ds(128, 16)] = tail
                for j in range(8):
                    lane = j // 2 if layer == 1 else 8
                    bc = jnp.full((16,), exrow[lane], jnp.float32)
                    msg_v[i, pl.ds(j * 16, 16)] = (
                        h_v[b][i, pl.ds(j * 16, 16)] * bc)

            pltpu.sync_copy(msg_v, acc_sh.at[dst_v[b]], add=True)

        fetch(0, 0)

        @pl.loop(0, bpt // 2)
        def _pair(p):
            fetch(1, 2 * p + 1)
            finish(0)

            @pl.when(2 * p + 2 < bpt)
            def _():
                fetch(0, 2 * p + 2)

            finish(1)

        plsc.subcore_barrier()
        pltpu.sync_copy(acc_sh.at[pl.ds(s * rps, rps)],
                        out_hbm.at[c].at[pl.ds(s * rps, rps)])

    return sc_pass


_sc_pass1 = _make_sc_pass(1)
_sc_pass2 = _make_sc_pass(2)


def kernel(x, edge_index, edge_attr, W1, a_src1, a_dst1, We1, ae1, b1,
           W2, a_src2, a_dst2, We2, ae2, b2):
    src, dst = edge_index[0], edge_index[1]

    As1 = _blockdiag(a_src1)
    Ad1 = _blockdiag(a_dst1)
    Ae1 = _blockdiag(ae1)
    As2 = a_src2.reshape(OUT, 1)
    Ad2 = a_dst2.reshape(OUT, 1)
    Ae2 = ae2.reshape(OUT, 1)

    srcp = jnp.concatenate([src, jnp.zeros((EP - E,), jnp.int32)])
    dstp = jnp.concatenate([dst, jnp.full((EP - E,), N, jnp.int32)])
    eap = jnp.concatenate(
        [edge_attr, jnp.zeros((EP - E, EDIM), jnp.float32)], axis=0)

    et = _edge_call(eap, We1, Ae1, We2, Ae2)
    h1, ts1, td1 = _node1_call(x, W1, As1, Ad1)

    zeros_acc = jnp.zeros((NPAD, ACCW), jnp.float32)
    padn = lambda t: jnp.concatenate(
        [t, jnp.zeros((NPAD - N, t.shape[1]), t.dtype)], axis=0)

    acc1 = _sc_pass1(srcp, dstp, et, padn(ts1), padn(td1), padn(h1),
                     zeros_acc)
    h2, ts2, td2, aux = _mid_call(acc1[0, :N], acc1[1, :N], h1, ts1, td1,
                                  W2, As2, Ad2, b1.reshape(1, -1))

    acc2 = _sc_pass2(srcp, dstp, et, padn(ts2), padn(td2), padn(h2),
                     zeros_acc)
    out = _final_call(acc2[0, :N], acc2[1, :N], h2, aux, b2.reshape(1, -1))
    return out


# speedup vs baseline: 15.6633x; 1.0044x over previous
import functools

import jax
import jax.numpy as jnp
from jax import lax
from jax.experimental import pallas as pl
from jax.experimental.pallas import tpu as pltpu
from jax.experimental.pallas import tpu_sc as plsc

N = 10000
E = 320000
DIN = 128
EDIM = 16
HID = 32
H1 = 4
OUT = 128

NC = 2
NS = 16
NW = NC * NS
B = 64
EP = ((E + NW * B * 2 - 1) // (NW * B * 2)) * (NW * B * 2)
NPAD = ((N + 1 + NS * 8 - 1) // (NS * 8)) * (NS * 8)
ACCW = 144
BM = 1000


def _blockdiag(a):
    h, c = a.shape
    out = jnp.zeros((h * c, h), jnp.float32)
    return out.at[jnp.arange(h * c), jnp.repeat(jnp.arange(h), c)].set(
        a.reshape(-1))


def _node1_body(x_ref, w1_ref, as_ref, ad_ref, h_ref, ts_ref, td_ref):
    h = jnp.dot(x_ref[...], w1_ref[...], preferred_element_type=jnp.float32)
    h_ref[...] = h
    s = jnp.dot(h, as_ref[...], preferred_element_type=jnp.float32)
    d = jnp.dot(h, ad_ref[...], preferred_element_type=jnp.float32)
    z = jnp.zeros((s.shape[0], 12), jnp.float32)
    ts_ref[...] = jnp.concatenate([s, z], axis=1)
    td_ref[...] = jnp.concatenate([d, z], axis=1)


def _node1_call(x, W1, As1, Ad1):
    nb = N // BM
    full = lambda shape: pl.BlockSpec(shape, lambda i: (0, 0))
    row = lambda w: pl.BlockSpec((BM, w), lambda i: (i, 0))
    return pl.pallas_call(
        _node1_body,
        grid=(nb,),
        in_specs=[row(DIN), full((DIN, H1 * HID)), full((H1 * HID, H1)),
                  full((H1 * HID, H1))],
        out_specs=[row(H1 * HID), row(16), row(16)],
        out_shape=[jax.ShapeDtypeStruct((N, H1 * HID), jnp.float32),
                   jax.ShapeDtypeStruct((N, 16), jnp.float32),
                   jax.ShapeDtypeStruct((N, 16), jnp.float32)],
    )(x, W1, As1, Ad1)


def _edge_body(ea_ref, we1_ref, ae1_ref, we2_ref, ae2_ref, et_ref):
    ve1 = jnp.dot(we1_ref[...], ae1_ref[...],
                  preferred_element_type=jnp.float32)
    ve2 = jnp.dot(we2_ref[...], ae2_ref[...],
                  preferred_element_type=jnp.float32)
    ea = ea_ref[...]
    e1 = jnp.dot(ea, ve1, preferred_element_type=jnp.float32)
    e2 = jnp.dot(ea, ve2, preferred_element_type=jnp.float32)
    bm = ea.shape[0]
    one = jnp.ones((bm, 1), jnp.float32)
    z6 = jnp.zeros((bm, 6), jnp.float32)
    et_ref[...] = jnp.concatenate([e1, e1, e2, one, z6], axis=1)


def _edge_call(eap, We1, Ae1, We2, Ae2):
    bm = NW * B
    nb = EP // bm
    full = lambda shape: pl.BlockSpec(shape, lambda i: (0, 0))
    return pl.pallas_call(
        _edge_body,
        grid=(nb,),
        in_specs=[pl.BlockSpec((bm, EDIM), lambda i: (i, 0)),
                  full((EDIM, H1 * HID)), full((H1 * HID, H1)),
                  full((EDIM, OUT)), full((OUT, 1))],
        out_specs=pl.BlockSpec((bm, 16), lambda i: (i, 0)),
        out_shape=jax.ShapeDtypeStruct((EP, 16), jnp.float32),
    )(eap, We1, Ae1, We2, Ae2)


def _lrelu(v):
    return jnp.where(v >= 0, v, 0.2 * v)


def _mid_body(a0_ref, a1_ref, h1_ref, ts_ref, td_ref, w2_ref, as2_ref,
              ad2_ref, b1_ref, h2_ref, ts2_ref, td2_ref, aux_ref):
    acc = a0_ref[...] + a1_ref[...]
    den1 = acc[:, 128:132]
    se1 = acc[:, 132:136]
    se2 = acc[:, 136:137]
    deg = acc[:, 137:138]
    mdeg = jnp.maximum(deg, 1.0)
    s1 = ts_ref[...][:, 0:4]
    d1 = td_ref[...][:, 0:4]
    exs1 = jnp.exp(_lrelu(s1 + d1 + se1 / mdeg))
    dent = den1 + exs1 + 1e-16
    h1 = h1_ref[...]
    cols = []
    for hh in range(H1):
        sl = slice(hh * HID, (hh + 1) * HID)
        cols.append((acc[:, sl] + exs1[:, hh:hh + 1] * h1[:, sl])
                    / dent[:, hh:hh + 1])
    out1 = jnp.concatenate(cols, axis=1) + b1_ref[...]
    h2in = jnp.maximum(out1, 0.0)
    h2 = jnp.dot(h2in, w2_ref[...], preferred_element_type=jnp.float32)
    h2_ref[...] = h2
    s2 = jnp.dot(h2, as2_ref[...], preferred_element_type=jnp.float32)
    d2 = jnp.dot(h2, ad2_ref[...], preferred_element_type=jnp.float32)
    exs2 = jnp.exp(_lrelu(s2 + d2 + se2 / mdeg))
    bm = acc.shape[0]
    z8 = jnp.zeros((bm, 8), jnp.float32)
    z7 = jnp.zeros((bm, 7), jnp.float32)
    ts2_ref[...] = jnp.concatenate([z8, s2, z7], axis=1)
    td2_ref[...] = jnp.concatenate([z8, d2, z7], axis=1)
    aux_ref[...] = jnp.concatenate([exs2, z8, z7], axis=1)


def _mid_call(acc0, acc1, h1, ts, td, W2, As2, Ad2, b1row):
    nb = N // BM
    full = lambda shape: pl.BlockSpec(shape, lambda i: (0, 0))
    row = lambda w: pl.BlockSpec((BM, w), lambda i: (i, 0))
    return pl.pallas_call(
        _mid_body,
        grid=(nb,),
        in_specs=[row(ACCW), row(ACCW), row(H1 * HID), row(16), row(16),
                  full((H1 * HID, OUT)), full((OUT, 1)), full((OUT, 1)),
                  full((1, H1 * HID))],
        out_specs=[row(OUT), row(16), row(16), row(16)],
        out_shape=[jax.ShapeDtypeStruct((N, OUT), jnp.float32),
                   jax.ShapeDtypeStruct((N, 16), jnp.float32),
                   jax.ShapeDtypeStruct((N, 16), jnp.float32),
                   jax.ShapeDtypeStruct((N, 16), jnp.float32)],
    )(acc0, acc1, h1, ts, td, W2, As2, Ad2, b1row)


def _final_body(a0_ref, a1_ref, h2_ref, aux_ref, b2_ref, out_ref):
    acc = a0_ref[...] + a1_ref[...]
    den2 = acc[:, 136:137]
    exs2 = aux_ref[...][:, 0:1]
    h2 = h2_ref[...]
    out_ref[...] = ((acc[:, 0:128] + exs2 * h2)
                    / (den2 + exs2 + 1e-16)) + b2_ref[...]


def _final_call(acc0, acc1, h2, aux, b2row):
    nb = N // BM
    full = lambda shape: pl.BlockSpec(shape, lambda i: (0, 0))
    row = lambda w: pl.BlockSpec((BM, w), lambda i: (i, 0))
    return pl.pallas_call(
        _final_body,
        grid=(nb,),
        in_specs=[row(ACCW), row(ACCW), row(OUT), row(16), full((1, OUT))],
        out_specs=row(OUT),
        out_shape=jax.ShapeDtypeStruct((N, OUT), jnp.float32),
    )(acc0, acc1, h2, aux, b2row)


def _make_sc_pass(layer):
    mesh = plsc.VectorSubcoreMesh(core_axis_name="c", subcore_axis_name="s")
    nblocks = EP // B
    bpt = nblocks // NW
    rps = NPAD // NS

    @functools.partial(
        pl.kernel,
        out_type=jax.ShapeDtypeStruct((NC, NPAD, ACCW), jnp.float32),
        mesh=mesh,
        compiler_params=pltpu.CompilerParams(use_tc_tiling_on_sc=False),
        scratch_types=[
            pltpu.VMEM((B,), jnp.int32), pltpu.VMEM((B,), jnp.int32),
            pltpu.VMEM((B,), jnp.int32), pltpu.VMEM((B,), jnp.int32),
            pltpu.VMEM((B, 16), jnp.float32), pltpu.VMEM((B, 16), jnp.float32),
            pltpu.VMEM((B, 16), jnp.float32), pltpu.VMEM((B, 16), jnp.float32),
            pltpu.VMEM((B, 16), jnp.float32), pltpu.VMEM((B, 16), jnp.float32),
            pltpu.VMEM((B, 128), jnp.float32),
            pltpu.VMEM((B, 128), jnp.float32),
            pltpu.VMEM((B, ACCW), jnp.float32),
            pltpu.VMEM_SHARED((NPAD, ACCW), jnp.float32),
            pltpu.SemaphoreType.DMA, pltpu.SemaphoreType.DMA,
            pltpu.SemaphoreType.DMA, pltpu.SemaphoreType.DMA,
            pltpu.SemaphoreType.DMA, pltpu.SemaphoreType.DMA,
        ],
    )
    def sc_pass(src_hbm, dst_hbm, et_hbm, tabs_hbm, tabd_hbm, h_hbm, zero_hbm,
                out_hbm, src0, src1, dst0, dst1, s0, s1, d0, d1, et0, et1,
                h0, h1b, msg_v, acc_sh, ss0, ss1, sd0, sd1, sh0, sh1):
        c = lax.axis_index("c")
        s = lax.axis_index("s")
        wid = s * NC + c
        src_v = [src0, src1]
        dst_v = [dst0, dst1]
        s_v = [s0, s1]
        d_v = [d0, d1]
        et_v = [et0, et1]
        h_v = [h0, h1b]
        sem_s = [ss0, ss1]
        sem_d = [sd0, sd1]
        sem_h = [sh0, sh1]

        pltpu.sync_copy(zero_hbm.at[pl.ds(s * rps, rps)],
                        acc_sh.at[pl.ds(s * rps, rps)])
        plsc.subcore_barrier()

        lanes = lax.iota(jnp.int32, 16)

        def fetch(b, bi):
            base = (bi * NW + wid) * B
            pltpu.sync_copy(src_hbm.at[pl.ds(base, B)], src_v[b])
            pltpu.sync_copy(dst_hbm.at[pl.ds(base, B)], dst_v[b])
            pltpu.sync_copy(et_hbm.at[pl.ds(base, B)], et_v[b])
            pltpu.make_async_copy(tabs_hbm.at[src_v[b]], s_v[b],
                                  sem_s[b]).start()
            pltpu.make_async_copy(tabd_hbm.at[dst_v[b]], d_v[b],
                                  sem_d[b]).start()
            pltpu.make_async_copy(h_hbm.at[src_v[b]], h_v[b],
                                  sem_h[b]).start()

        def finish(b):
            pltpu.make_async_copy(tabs_hbm.at[src_v[b]], s_v[b],
                                  sem_s[b]).wait()
            pltpu.make_async_copy(tabd_hbm.at[dst_v[b]], d_v[b],
                                  sem_d[b]).wait()
            pltpu.make_async_copy(h_hbm.at[src_v[b]], h_v[b],
                                  sem_h[b]).wait()

            def one_edge(i):
                arow = s_v[b][i] + d_v[b][i] + et_v[b][i]
                exrow = jnp.exp(_lrelu(arow))
                if layer == 1:
                    tail = jnp.where(lanes < 4, exrow, et_v[b][i])
                    bcs = [jnp.full((16,), exrow[h], jnp.float32)
                           for h in range(4)]
                else:
                    tail = jnp.where(lanes == 8, exrow,
                                     jnp.zeros((16,), jnp.float32))
                    bcs = [jnp.full((16,), exrow[8], jnp.float32)]
                msg_v[i, pl.ds(128, 16)] = tail
                for j in range(8):
                    bc = bcs[j // 2 if layer == 1 else 0]
                    msg_v[i, pl.ds(j * 16, 16)] = (
                        h_v[b][i, pl.ds(j * 16, 16)] * bc)

            @pl.loop(0, B, step=2)
            def _edge(i):
                one_edge(i)
                one_edge(i + 1)

            pltpu.sync_copy(msg_v, acc_sh.at[dst_v[b]], add=True)

        fetch(0, 0)

        @pl.loop(0, bpt // 2)
        def _pair(p):
            fetch(1, 2 * p + 1)
            finish(0)

            @pl.when(2 * p + 2 < bpt)
            def _():
                fetch(0, 2 * p + 2)

            finish(1)

        plsc.subcore_barrier()
        pltpu.sync_copy(acc_sh.at[pl.ds(s * rps, rps)],
                        out_hbm.at[c].at[pl.ds(s * rps, rps)])

    return sc_pass


_sc_pass1 = _make_sc_pass(1)
_sc_pass2 = _make_sc_pass(2)


def kernel(x, edge_index, edge_attr, W1, a_src1, a_dst1, We1, ae1, b1,
           W2, a_src2, a_dst2, We2, ae2, b2):
    src, dst = edge_index[0], edge_index[1]

    As1 = _blockdiag(a_src1)
    Ad1 = _blockdiag(a_dst1)
    Ae1 = _blockdiag(ae1)
    As2 = a_src2.reshape(OUT, 1)
    Ad2 = a_dst2.reshape(OUT, 1)
    Ae2 = ae2.reshape(OUT, 1)

    srcp = jnp.concatenate([src, jnp.zeros((EP - E,), jnp.int32)])
    dstp = jnp.concatenate([dst, jnp.full((EP - E,), N, jnp.int32)])
    eap = jnp.concatenate(
        [edge_attr, jnp.zeros((EP - E, EDIM), jnp.float32)], axis=0)

    et = _edge_call(eap, We1, Ae1, We2, Ae2)
    h1, ts1, td1 = _node1_call(x, W1, As1, Ad1)

    zeros_acc = jnp.zeros((NPAD, ACCW), jnp.float32)
    padn = lambda t: jnp.concatenate(
        [t, jnp.zeros((NPAD - N, t.shape[1]), t.dtype)], axis=0)

    acc1 = _sc_pass1(srcp, dstp, et, padn(ts1), padn(td1), padn(h1),
                     zeros_acc)
    h2, ts2, td2, aux = _mid_call(acc1[0, :N], acc1[1, :N], h1, ts1, td1,
                                  W2, As2, Ad2, b1.reshape(1, -1))

    acc2 = _sc_pass2(srcp, dstp, et, padn(ts2), padn(td2), padn(h2),
                     zeros_acc)
    out = _final_call(acc2[0, :N], acc2[1, :N], h2, aux, b2.reshape(1, -1))
    return out
